# Initial kernel scaffold; baseline (speedup 1.0000x reference)
import jax
import jax.numpy as jnp
from jax.experimental import pallas as pl


def kernel(nodes, edges, graph_globals, senders, receivers, batch, edgepos, eW0, eb0, eW1, eb1, eW2, eb2, eW3, eb3, nW0, nb0, nW1, nb1, nW2, nb2, nW3, nb3, gW0, gb0, gW1, gb1, gW2, gb2, gW3, gb3):
    raise NotImplementedError("write your pallas kernel here")


# speedup vs baseline: 6.3294x; 6.3294x over previous
import jax
import jax.numpy as jnp
from jax import lax
from jax.experimental import pallas as pl
from jax.experimental.pallas import tpu as pltpu
from jax.experimental.pallas import tpu_sc as plsc

N = 10000
E = 320000
G = 8
D_NODE = 128
D_EDGE = 16
D_GLOB = 128
H = 128
E_OUT = 16

_NC = 2
_NS = 16
_NW = _NC * _NS
_CH = 80
_NPAD = 10240
_NPS = _NPAD // _NS

_NSLICE = 5
_ES = E // _NSLICE
_EPW = _ES // _NW
_NCH = _EPW // _CH
_ETILE = 2560


def _ln_relu(x):
    n = x.shape[-1]
    xb = x.astype(jnp.bfloat16)
    ones_b = jnp.full((n, n), 1.0 / n, dtype=jnp.bfloat16)
    m = jnp.dot(xb, ones_b, preferred_element_type=jnp.float32)
    ex2 = jnp.dot(xb * xb, ones_b, preferred_element_type=jnp.float32)
    v = ex2 - m * m
    return jnp.maximum((x - m) * lax.rsqrt(v + 1e-5), 0.0)


def _proj_body(nodes_ref, ws_ref, wr_ref, gg_ref, wqe_ref, eb0_ref, wqn_ref,
               nb0_ref, ps_ref, pr_ref, qe_ref, qn_ref):
    x = nodes_ref[...]
    ps_ref[...] = jnp.dot(x, ws_ref[...], preferred_element_type=jnp.float32)
    pr_ref[...] = jnp.dot(x, wr_ref[...], preferred_element_type=jnp.float32)

    @pl.when(pl.program_id(0) == 0)
    def _():
        gg = gg_ref[...]
        qe_ref[...] = jnp.dot(gg, wqe_ref[...],
                              preferred_element_type=jnp.float32) + eb0_ref[...]
        qn_ref[...] = jnp.dot(gg, wqn_ref[...],
                              preferred_element_type=jnp.float32) + nb0_ref[...]


def _proj(nodes, ws, wr, gg, wqe, eb0, wqn, nb0, interpret=False):
    tile = 1000
    rep = lambda i: (0, 0)
    return pl.pallas_call(
        _proj_body,
        grid=(N // tile,),
        in_specs=[
            pl.BlockSpec((tile, D_NODE), lambda i: (i, 0)),
            pl.BlockSpec((D_NODE, H), rep),
            pl.BlockSpec((D_NODE, H), rep),
            pl.BlockSpec((G, D_GLOB), rep),
            pl.BlockSpec((D_GLOB, H), rep),
            pl.BlockSpec((1, H), rep),
            pl.BlockSpec((D_GLOB, H), rep),
            pl.BlockSpec((1, H), rep),
        ],
        out_specs=[
            pl.BlockSpec((tile, H), lambda i: (i, 0)),
            pl.BlockSpec((tile, H), lambda i: (i, 0)),
            pl.BlockSpec((G, H), rep),
            pl.BlockSpec((G, H), rep),
        ],
        out_shape=[
            jax.ShapeDtypeStruct((N, H), jnp.float32),
            jax.ShapeDtypeStruct((N, H), jnp.float32),
            jax.ShapeDtypeStruct((G, H), jnp.float32),
            jax.ShapeDtypeStruct((G, H), jnp.float32),
        ],
        interpret=interpret,
    )(nodes, ws, wr, gg, wqe, eb0, wqn, nb0)


def _sc_gather_body(ps_hbm, pr_hbm, snd_hbm, rcv_hbm, out_hbm,
                    idx_s, idx_r, bs0, br0, bs1, br1,
                    sem_s0, sem_r0, sem_s1, sem_r1):
    wid = lax.axis_index("s") * _NC + lax.axis_index("c")
    base = wid * _EPW
    pltpu.sync_copy(snd_hbm.at[pl.ds(base, _EPW)], idx_s)
    pltpu.sync_copy(rcv_hbm.at[pl.ds(base, _EPW)], idx_r)

    def fire(ci, bs, br, ss, sr):
        pltpu.async_copy(ps_hbm.at[idx_s.at[pl.ds(ci * _CH, _CH)]], bs, ss)
        pltpu.async_copy(pr_hbm.at[idx_r.at[pl.ds(ci * _CH, _CH)]], br, sr)

    def drain_process(ci, bs, br, ss, sr):
        pltpu.make_async_copy(
            ps_hbm.at[idx_s.at[pl.ds(ci * _CH, _CH)]], bs, ss).wait()
        pltpu.make_async_copy(
            pr_hbm.at[idx_r.at[pl.ds(ci * _CH, _CH)]], br, sr).wait()

        def row(k, _):
            for j in range(H // 16):
                sl = pl.ds(j * 16, 16)
                bs[k, sl] = bs[k, sl] + br[k, sl]
            return 0

        lax.fori_loop(0, _CH, row, 0)
        pltpu.sync_copy(bs, out_hbm.at[pl.ds(base + ci * _CH, _CH)])

    fire(0, bs0, br0, sem_s0, sem_r0)

    def grp(g, _):
        c0 = g * 2
        c1 = c0 + 1

        @pl.when(c1 < _NCH)
        def _():
            fire(c1, bs1, br1, sem_s1, sem_r1)

        drain_process(c0, bs0, br0, sem_s0, sem_r0)

        @pl.when(c1 < _NCH)
        def _():
            @pl.when(c1 + 1 < _NCH)
            def _():
                fire(c1 + 1, bs0, br0, sem_s0, sem_r0)

            drain_process(c1, bs1, br1, sem_s1, sem_r1)

        return 0

    lax.fori_loop(0, (_NCH + 1) // 2, grp, 0)


def _sc_gather(ps, pr, snd, rcv):
    fn = pl.kernel(
        _sc_gather_body,
        mesh=plsc.VectorSubcoreMesh(core_axis_name="c", subcore_axis_name="s"),
        out_type=jax.ShapeDtypeStruct((_ES, H), jnp.float32),
        scratch_types=[
            pltpu.VMEM((_EPW,), jnp.int32),
            pltpu.VMEM((_EPW,), jnp.int32),
            pltpu.VMEM((_CH, H), jnp.float32),
            pltpu.VMEM((_CH, H), jnp.float32),
            pltpu.VMEM((_CH, H), jnp.float32),
            pltpu.VMEM((_CH, H), jnp.float32),
            pltpu.SemaphoreType.DMA,
            pltpu.SemaphoreType.DMA,
            pltpu.SemaphoreType.DMA,
            pltpu.SemaphoreType.DMA,
        ],
    )
    return fn(ps, pr, snd, rcv)


def _edge_body(h0_ref, edges_ref, epos_ref, we_ref, qe_ref, w1_ref, b1_ref,
               w2_ref, b2_ref, w3_ref, b3_ref, eout_ref, eagg_ref):
    epos = epos_ref[...][0]
    oht = (lax.broadcasted_iota(jnp.int32, (G, epos.shape[1]), 0)
           == epos).astype(jnp.float32)
    x = (h0_ref[...]
         + jnp.dot(edges_ref[...].astype(jnp.bfloat16), we_ref[...],
                   preferred_element_type=jnp.float32)
         + lax.dot_general(oht, qe_ref[...], (((0,), (0,)), ((), ())),
                           preferred_element_type=jnp.float32))
    t = _ln_relu(x).astype(jnp.bfloat16)
    t = jnp.dot(t, w1_ref[...], preferred_element_type=jnp.float32) + b1_ref[...]
    t = _ln_relu(t).astype(jnp.bfloat16)
    t = jnp.dot(t, w2_ref[...], preferred_element_type=jnp.float32) + b2_ref[...]
    t = _ln_relu(t).astype(jnp.bfloat16)
    eo = jnp.dot(t, w3_ref[...], preferred_element_type=jnp.float32) + b3_ref[...]
    eout_ref[...] = eo

    @pl.when(pl.program_id(0) == 0)
    def _():
        eagg_ref[...] = jnp.zeros_like(eagg_ref)

    eagg_ref[...] += jnp.dot(oht, eo, preferred_element_type=jnp.float32)


def _edge_mlp(h0, edges, epos3, we, qe, w1, b1, w2, b2, w3, b3,
              interpret=False):
    tile = _ETILE
    grid = (_ES // tile,)
    rep = lambda i: (0, 0)
    return pl.pallas_call(
        _edge_body,
        grid=grid,
        in_specs=[
            pl.BlockSpec((tile, H), lambda i: (i, 0)),
            pl.BlockSpec((tile, D_EDGE), lambda i: (i, 0)),
            pl.BlockSpec((1, 1, tile), lambda i: (i, 0, 0)),
            pl.BlockSpec((D_EDGE, H), rep),
            pl.BlockSpec((G, H), rep),
            pl.BlockSpec((H, H), rep),
            pl.BlockSpec((1, H), rep),
            pl.BlockSpec((H, H), rep),
            pl.BlockSpec((1, H), rep),
            pl.BlockSpec((H, E_OUT), rep),
            pl.BlockSpec((1, E_OUT), rep),
        ],
        out_specs=[
            pl.BlockSpec((tile, E_OUT), lambda i: (i, 0)),
            pl.BlockSpec((G, E_OUT), rep),
        ],
        out_shape=[
            jax.ShapeDtypeStruct((_ES, E_OUT), jnp.float32),
            jax.ShapeDtypeStruct((G, E_OUT), jnp.float32),
        ],
        interpret=interpret,
    )(h0, edges, epos3, we, qe, w1, b1, w2, b2, w3, b3)


def _sc_scatter_body(eout_hbm, rcv2_hbm, zeros_hbm, zrow_hbm, out_hbm,
                     aggsp, ebuf, vbuf, idx2):
    c = lax.axis_index("c")
    s = lax.axis_index("s")
    wid = s * _NC + c
    base = wid * _EPW
    pltpu.sync_copy(rcv2_hbm.at[wid], idx2)
    pltpu.sync_copy(zeros_hbm.at[pl.ds(s * _NPS, _NPS)],
                    aggsp.at[pl.ds(s * _NPS, _NPS)])
    pltpu.sync_copy(zrow_hbm, vbuf)
    plsc.subcore_barrier()

    def chunk(ci, _):
        pltpu.sync_copy(eout_hbm.at[pl.ds(base + ci * _CH, _CH)], ebuf)

        def row(k, _):
            vbuf[k, pl.ds(0, E_OUT)] = ebuf[k, pl.ds(0, E_OUT)]
            return 0

        lax.fori_loop(0, _CH, row, 0)
        pltpu.sync_copy(vbuf, aggsp.at[idx2.at[ci]], add=True)
        return 0

    lax.fori_loop(0, _NCH, chunk, 0)
    plsc.subcore_barrier()
    pltpu.sync_copy(aggsp.at[pl.ds(s * _NPS, _NPS)],
                    out_hbm.at[c, pl.ds(s * _NPS, _NPS)])


def _sc_scatter(e_out, rcv, zeros, zrow):
    fn = pl.kernel(
        _sc_scatter_body,
        mesh=plsc.VectorSubcoreMesh(core_axis_name="c", subcore_axis_name="s"),
        out_type=jax.ShapeDtypeStruct((_NC, _NPAD, 128), jnp.float32),
        scratch_types=[
            pltpu.VMEM_SHARED((_NPAD, 128), jnp.float32),
            pltpu.VMEM((_CH, E_OUT), jnp.float32),
            pltpu.VMEM((_CH, 128), jnp.float32),
            pltpu.VMEM((_NCH, _CH), jnp.int32),
        ],
    )
    return fn(e_out, rcv.reshape(_NW, _NCH, _CH), zeros, zrow)


def _node_body(nodes_ref, a0_ref, a1_ref, a2_ref, a3_ref, a4_ref, batch_ref,
               w0a_ref, w0b_ref, qn_ref,
               w1_ref, b1_ref, w2_ref, b2_ref, w3_ref, b3_ref,
               eagg_ref, gg_ref, gw0a_ref, gw0b_ref, gw0c_ref, gb0_ref,
               gw1_ref, gb1_ref, gw2_ref, gb2_ref, gw3_ref, gb3_ref,
               nout_ref, gout_ref, nagg_acc):
    i = pl.program_id(0)
    b = batch_ref[...][0]
    oht = (lax.broadcasted_iota(jnp.int32, (G, b.shape[1]), 0)
           == b).astype(jnp.float32)
    agg = jnp.zeros((nodes_ref.shape[0], E_OUT), jnp.float32)
    for aref in (a0_ref, a1_ref, a2_ref, a3_ref, a4_ref):
        a2v = aref[...]
        agg = agg + a2v[0, :, :E_OUT] + a2v[1, :, :E_OUT]
    x = (jnp.dot(nodes_ref[...].astype(jnp.bfloat16), w0a_ref[...],
                 preferred_element_type=jnp.float32)
         + jnp.dot(agg.astype(jnp.bfloat16), w0b_ref[...],
                   preferred_element_type=jnp.float32)
         + lax.dot_general(oht, qn_ref[...], (((0,), (0,)), ((), ())),
                           preferred_element_type=jnp.float32))
    t = _ln_relu(x).astype(jnp.bfloat16)
    t = jnp.dot(t, w1_ref[...], preferred_element_type=jnp.float32) + b1_ref[...]
    t = _ln_relu(t).astype(jnp.bfloat16)
    t = jnp.dot(t, w2_ref[...], preferred_element_type=jnp.float32) + b2_ref[...]
    t = _ln_relu(t).astype(jnp.bfloat16)
    no = jnp.dot(t, w3_ref[...], preferred_element_type=jnp.float32) + b3_ref[...]
    nout_ref[...] = no

    @pl.when(i == 0)
    def _():
        nagg_acc[...] = jnp.zeros_like(nagg_acc)

    nagg_acc[...] += jnp.dot(oht, no, preferred_element_type=jnp.float32)

    @pl.when(i == pl.num_programs(0) - 1)
    def _():
        eagg = jnp.sum(eagg_ref[...], axis=0)
        gx = (jnp.dot(nagg_acc[...], gw0a_ref[...],
                      preferred_element_type=jnp.float32)
              + jnp.dot(eagg, gw0b_ref[...],
                        preferred_element_type=jnp.float32)
              + jnp.dot(gg_ref[...], gw0c_ref[...],
                        preferred_element_type=jnp.float32)
              + gb0_ref[...])
        gt = _ln_relu(gx)
        gt = jnp.dot(gt, gw1_ref[...],
                     preferred_element_type=jnp.float32) + gb1_ref[...]
        gt = _ln_relu(gt)
        gt = jnp.dot(gt, gw2_ref[...],
                     preferred_element_type=jnp.float32) + gb2_ref[...]
        gt = _ln_relu(gt)
        gout_ref[...] = jnp.dot(gt, gw3_ref[...],
                                preferred_element_type=jnp.float32) + gb3_ref[...]


def _node_global(nodes, aggs, batch3, w0a, w0b, qn, w1, b1, w2, b2, w3, b3,
                 eagg, gg, gw0a, gw0b, gw0c, gb0, gw1, gb1, gw2, gb2, gw3, gb3,
                 interpret=False):
    tile = 1000
    grid = (N // tile,)
    rep = lambda i: (0, 0)
    agg_spec = pl.BlockSpec((_NC, tile, 128), lambda i: (0, i, 0))
    return pl.pallas_call(
        _node_body,
        grid=grid,
        in_specs=[
            pl.BlockSpec((tile, D_NODE), lambda i: (i, 0)),
            agg_spec, agg_spec, agg_spec, agg_spec, agg_spec,
            pl.BlockSpec((1, 1, tile), lambda i: (i, 0, 0)),
            pl.BlockSpec((D_NODE, H), rep),
            pl.BlockSpec((E_OUT, H), rep),
            pl.BlockSpec((G, H), rep),
            pl.BlockSpec((H, H), rep),
            pl.BlockSpec((1, H), rep),
            pl.BlockSpec((H, H), rep),
            pl.BlockSpec((1, H), rep),
            pl.BlockSpec((H, H), rep),
            pl.BlockSpec((1, H), rep),
            pl.BlockSpec((_NSLICE, G, E_OUT), lambda i: (0, 0, 0)),
            pl.BlockSpec((G, D_GLOB), rep),
            pl.BlockSpec((H, H), rep),
            pl.BlockSpec((E_OUT, H), rep),
            pl.BlockSpec((D_GLOB, H), rep),
            pl.BlockSpec((1, H), rep),
            pl.BlockSpec((H, H), rep),
            pl.BlockSpec((1, H), rep),
            pl.BlockSpec((H, H), rep),
            pl.BlockSpec((1, H), rep),
            pl.BlockSpec((H, H), rep),
            pl.BlockSpec((1, H), rep),
        ],
        out_specs=[
            pl.BlockSpec((tile, H), lambda i: (i, 0)),
            pl.BlockSpec((G, H), rep),
        ],
        out_shape=[
            jax.ShapeDtypeStruct((N, H), jnp.float32),
            jax.ShapeDtypeStruct((G, H), jnp.float32),
        ],
        scratch_shapes=[pltpu.VMEM((G, H), jnp.float32)],
        interpret=interpret,
    )(nodes, *aggs, batch3, w0a, w0b, qn, w1, b1, w2, b2, w3, b3,
      eagg, gg, gw0a, gw0b, gw0c, gb0, gw1, gb1, gw2, gb2, gw3, gb3)


def kernel(nodes, edges, graph_globals, senders, receivers, batch, edgepos,
           eW0, eb0, eW1, eb1, eW2, eb2, eW3, eb3,
           nW0, nb0, nW1, nb1, nW2, nb2, nW3, nb3,
           gW0, gb0, gW1, gb1, gW2, gb2, gW3, gb3):
    w_e = eW0[:D_EDGE]
    w_s = eW0[D_EDGE:D_EDGE + D_NODE]
    w_r = eW0[D_EDGE + D_NODE:D_EDGE + 2 * D_NODE]
    w_qe = eW0[D_EDGE + 2 * D_NODE:]
    nw0a = nW0[:D_NODE]
    nw0b = nW0[D_NODE:D_NODE + E_OUT]
    w_qn = nW0[D_NODE + E_OUT:]
    gw0a = gW0[:H]
    gw0b = gW0[H:H + E_OUT]
    gw0c = gW0[H + E_OUT:]
    r2 = lambda v: v.reshape(1, -1)
    bf = lambda v: v.astype(jnp.bfloat16)

    ps, pr, qe, qn = _proj(nodes, w_s, w_r, graph_globals, w_qe, r2(eb0),
                           w_qn, r2(nb0))
    zeros = jnp.zeros((_NPAD, 128), jnp.float32)
    zrow = jnp.zeros((_CH, 128), jnp.float32)

    eouts, eaggs, aggs = [], [], []
    for si in range(_NSLICE):
        sl = slice(si * _ES, (si + 1) * _ES)
        h0 = _sc_gather(ps, pr, senders[sl], receivers[sl])
        e_out, eagg = _edge_mlp(h0, edges[sl],
                                edgepos[sl].reshape(_ES // _ETILE, 1, _ETILE),
                                bf(w_e), qe, bf(eW1), r2(eb1), bf(eW2),
                                r2(eb2), bf(eW3), r2(eb3))
        agg2 = _sc_scatter(e_out, receivers[sl], zeros, zrow)
        eouts.append(e_out)
        eaggs.append(eagg)
        aggs.append(agg2)

    n_out, g_out = _node_global(
        nodes, aggs, batch.reshape(N // 1000, 1, 1000),
        bf(nw0a), bf(nw0b), qn, bf(nW1), r2(nb1), bf(nW2), r2(nb2),
        bf(nW3), r2(nb3),
        jnp.stack(eaggs), graph_globals, gw0a, gw0b, gw0c, r2(gb0),
        gW1, r2(gb1), gW2, r2(gb2), gW3, r2(gb3))
    e_out_full = jnp.concatenate(eouts, axis=0)
    return e_out_full, n_out, g_out
